# baseline (speedup 1.0000x reference)
import functools

import jax
import jax.numpy as jnp
from jax import lax
from jax.experimental import pallas as pl
from jax.experimental.pallas import tpu as pltpu
from jax.experimental.pallas import tpu_sc as plsc

N_ENT = 100000
DIM = 128
GAMMA = 12.0

NC, NS, L = 2, 16, 16
CHUNK = 6144
N_CHUNKS = 18
N_TAB = CHUNK * N_CHUNKS
CPC = N_CHUNKS // NC
GRP = 128
RPT = CHUNK // NS
RBT = 2048
ZR = 32
BIG = 1 << 29
assert CHUNK % 2048 == 0 and N_CHUNKS % 2 == 0 and N_TAB % RBT == 0
assert RPT % ZR == 0 and CHUNK * N_CHUNKS >= N_ENT

_mesh = functools.partial(
    plsc.VectorSubcoreMesh, core_axis_name="c", subcore_axis_name="s",
    num_cores=NC, num_subcores=NS)


def _make_seg_sum(blk, n_blocks):
  share = blk * n_blocks
  g_max = blk // GRP + 2
  assert blk % (2 * L) == 0

  def body(table, src_h, dst_h, sums_o, cnts_o,
           srcblk, dstblk, srcflat, dst2d, rows_a, rows_b, ones_v,
           zrows, zcnt, acc, cnt_acc, sem_a, sem_b, sem_c):
    c = lax.axis_index("c")
    s = lax.axis_index("s")
    wid = s * NC + c

    zv = jnp.zeros((L,), jnp.float32)

    def fill_ones(j, _):
      ones_v[pl.ds(j * L, L)] = jnp.full((L,), 1.0, jnp.float32)
      return 0
    lax.fori_loop(0, GRP // L, fill_ones, 0)

    def fill_zrows(j, _):
      zrows[j // 8, pl.ds((j % 8) * L, L)] = zv
      return 0
    lax.fori_loop(0, ZR * DIM // L, fill_zrows, 0)

    def fill_zcnt(j, _):
      zcnt[pl.ds(j * L, L)] = zv
      return 0
    lax.fori_loop(0, RPT // L, fill_zcnt, 0)

    def zero_acc():
      for j in range(RPT // ZR):
        pltpu.sync_copy(zrows, acc.at[pl.ds(s * RPT + j * ZR, ZR)])
      pltpu.sync_copy(zrows.at[pl.ds(0, 8)], acc.at[pl.ds(CHUNK + s * 8, 8)])
      pltpu.sync_copy(zcnt, cnt_acc.at[pl.ds(s * RPT, RPT)])

      @pl.when(s == NS - 1)
      def _():
        pltpu.sync_copy(zcnt.at[pl.ds(0, GRP)],
                        cnt_acc.at[pl.ds(CHUNK, GRP)])

    zero_acc()
    plsc.subcore_barrier()

    iota = lax.iota(jnp.int32, L)
    pad_src = iota + wid * L
    pad_dst = jnp.full((L,), CHUNK, jnp.int32) + s * 8

    def start_g(g, buf, sm):
      pltpu.async_copy(
          table.at[srcflat.at[pl.ds(g * GRP, GRP)]], buf, sm)

    def wait_g(buf, sm):
      pltpu.make_async_copy(
          table.at[srcflat.at[pl.ds(0, GRP)]], buf, sm).wait()

    def scat(g, buf):
      pltpu.async_copy(ones_v, cnt_acc.at[dst2d.at[g]], sem_c, add=True)
      pltpu.sync_copy(buf, acc.at[dst2d.at[g]], add=True)

    def drain_cnt(ng):
      def one(g, _):
        pltpu.make_async_copy(ones_v, cnt_acc.at[dst2d.at[0]], sem_c).wait()
        return 0
      lax.fori_loop(0, ng, one, 0)

    def flush(ng):
      @pl.when(ng > 0)
      def _():
        start_g(0, rows_a, sem_a)

      def body2(p, _):
        g0 = 2 * p
        g1 = g0 + 1

        @pl.when(g1 < ng)
        def _():
          start_g(g1, rows_b, sem_b)
        wait_g(rows_a, sem_a)
        scat(g0, rows_a)

        @pl.when(g0 + 2 < ng)
        def _():
          start_g(g0 + 2, rows_a, sem_a)

        @pl.when(g1 < ng)
        def _():
          wait_g(rows_b, sem_b)
          scat(g1, rows_b)
        return 0
      lax.fori_loop(0, (ng + 1) // 2, body2, 0)
      drain_cnt(ng)

    for k in range(CPC):
      lo = (k * NC + c) * CHUNK
      hi = lo + CHUNK

      def block_body(b, rem):
        base_e = s * share + b * blk
        pltpu.sync_copy(src_h.at[pl.ds(base_e, blk)], srcblk)
        pltpu.sync_copy(dst_h.at[pl.ds(base_e, blk)], dstblk)

        def emit(nv, dv, sv):
          m = (dv >= lo) & (dv < hi)
          mi = m.astype(jnp.int32)
          pos = nv + plsc.cumsum(mi) - mi
          plsc.store_scatter(srcflat, [pos], sv, mask=m)
          plsc.store_scatter(dst2d, [pos >> 7, pos & (GRP - 1)], dv - lo,
                             mask=m)
          return nv + plsc.all_reduce_population_count(m)

        def scan2(i2, nv):
          base = i2 * 2 * L
          nv = emit(nv, dstblk[pl.ds(base, L)], srcblk[pl.ds(base, L)])
          nv = emit(nv, dstblk[pl.ds(base + L, L)], srcblk[pl.ds(base + L, L)])
          return nv

        nvec = lax.fori_loop(0, blk // (2 * L), scan2,
                             jnp.full((L,), rem, jnp.int32))
        n = jnp.max(nvec)
        ng = n >> 7
        flush(ng)
        for j in range(GRP // L):
          v = srcflat[pl.ds(ng * GRP + j * L, L)]
          srcflat[pl.ds(j * L, L)] = v
          w = dst2d[ng, pl.ds(j * L, L)]
          dst2d[0, pl.ds(j * L, L)] = w
        return n & (GRP - 1)

      rem = lax.fori_loop(0, n_blocks, block_body, jnp.int32(0))

      for j in range(GRP // L):
        idxp = rem + j * L + iota
        plsc.store_scatter(srcflat, [idxp], pad_src)
        plsc.store_scatter(dst2d, [idxp >> 7, idxp & (GRP - 1)], pad_dst)

      @pl.when(rem > 0)
      def _():
        start_g(0, rows_a, sem_a)
        wait_g(rows_a, sem_a)
        scat(0, rows_a)
        drain_cnt(1)

      plsc.subcore_barrier()
      pltpu.sync_copy(acc.at[pl.ds(s * RPT, RPT)],
                      sums_o.at[pl.ds(lo + s * RPT, RPT)])
      pltpu.sync_copy(cnt_acc.at[pl.ds(s * RPT, RPT)],
                      cnts_o.at[pl.ds(lo + s * RPT, RPT)])
      zero_acc()
      plsc.subcore_barrier()

  return pl.kernel(
      body,
      out_type=(jax.ShapeDtypeStruct((N_TAB, DIM), jnp.float32),
                jax.ShapeDtypeStruct((N_TAB,), jnp.float32)),
      mesh=_mesh(),
      compiler_params=pltpu.CompilerParams(needs_layout_passes=False),
      scratch_types=[
          pltpu.VMEM((blk,), jnp.int32),
          pltpu.VMEM((blk,), jnp.int32),
          pltpu.VMEM((blk + 2 * GRP,), jnp.int32),
          pltpu.VMEM((g_max, GRP), jnp.int32),
          pltpu.VMEM((GRP, DIM), jnp.float32),
          pltpu.VMEM((GRP, DIM), jnp.float32),
          pltpu.VMEM((GRP,), jnp.float32),
          pltpu.VMEM((ZR, DIM), jnp.float32),
          pltpu.VMEM((RPT,), jnp.float32),
          pltpu.VMEM_SHARED((CHUNK + GRP, DIM), jnp.float32),
          pltpu.VMEM_SHARED((CHUNK + GRP,), jnp.float32),
          pltpu.SemaphoreType.DMA,
          pltpu.SemaphoreType.DMA,
          pltpu.SemaphoreType.DMA,
      ],
  )


def _gather_kernel():
  nt = 16384 // (NC * NS)

  def body(e_h, r_h, s3_h, c3_h, hh, rh, th,
           eh_o, et_o, th_o, tt_o, rr_o, ch_o, ct_o,
           hbuf, rbuf, tbuf, rows_v, crow, sem):
    c = lax.axis_index("c")
    s = lax.axis_index("s")
    wid = s * NC + c
    base = wid * nt
    pltpu.sync_copy(hh.at[pl.ds(base, nt)], hbuf)
    pltpu.sync_copy(rh.at[pl.ds(base, nt)], rbuf)
    pltpu.sync_copy(th.at[pl.ds(base, nt)], tbuf)
    for g in range(nt // GRP):
      sl = pl.ds(base + g * GRP, GRP)
      gi = pl.ds(g * GRP, GRP)
      for tab, idxb, out in ((e_h, hbuf, eh_o), (e_h, tbuf, et_o),
                             (s3_h, hbuf, th_o), (s3_h, tbuf, tt_o),
                             (r_h, rbuf, rr_o)):
        pltpu.async_copy(tab.at[idxb.at[gi]], rows_v, sem).wait()
        pltpu.sync_copy(rows_v, out.at[sl])
      for idxb, out in ((hbuf, ch_o), (tbuf, ct_o)):
        pltpu.async_copy(c3_h.at[idxb.at[gi]], crow, sem).wait()
        pltpu.sync_copy(crow, out.at[sl])

  f32 = jnp.float32
  return pl.kernel(
      body,
      out_type=(jax.ShapeDtypeStruct((16384, DIM), f32),) * 5
      + (jax.ShapeDtypeStruct((16384,), f32),) * 2,
      mesh=_mesh(),
      compiler_params=pltpu.CompilerParams(needs_layout_passes=False),
      scratch_types=[
          pltpu.VMEM((nt,), jnp.int32),
          pltpu.VMEM((nt,), jnp.int32),
          pltpu.VMEM((nt,), jnp.int32),
          pltpu.VMEM((GRP, DIM), f32),
          pltpu.VMEM((GRP,), f32),
          pltpu.SemaphoreType.DMA,
      ],
  )


RB = 2048


def _mean_mm(sums, cnts, wt1, b1, wt2=None, b2=None):
  two = wt2 is not None

  def body(s_ref, c_ref, w1_ref, b1_ref, *rest):
    if two:
      w2_ref, b2_ref, o_ref = rest
    else:
      (o_ref,) = rest
    cnt = c_ref[...].reshape(RBT)
    den = jnp.where(cnt > 0, cnt, 1.0)
    x = s_ref[...] / den[:, None]
    y = lax.dot_general(x, w1_ref[...], (((1,), (0,)), ((), ())),
                        preferred_element_type=jnp.float32,
                        precision=lax.Precision.HIGHEST)
    y = jnp.maximum(y + b1_ref[...], 0.0)
    if two:
      y = lax.dot_general(y, w2_ref[...], (((1,), (0,)), ((), ())),
                          preferred_element_type=jnp.float32,
                          precision=lax.Precision.HIGHEST) + b2_ref[...]
    o_ref[...] = y

  n_blk = N_TAB // RBT
  in_specs = [
      pl.BlockSpec((RBT, DIM), lambda i: (i, 0)),
      pl.BlockSpec((RBT // DIM, DIM), lambda i: (i, 0)),
      pl.BlockSpec((DIM, DIM), lambda i: (0, 0)),
      pl.BlockSpec((1, DIM), lambda i: (0, 0)),
  ]
  args = [sums, cnts.reshape(N_TAB // DIM, DIM), wt1, b1.reshape(1, DIM)]
  if two:
    in_specs += [pl.BlockSpec((DIM, DIM), lambda i: (0, 0)),
                 pl.BlockSpec((1, DIM), lambda i: (0, 0))]
    args += [wt2, b2.reshape(1, DIM)]
  return pl.pallas_call(
      body, grid=(n_blk,), in_specs=in_specs,
      out_specs=pl.BlockSpec((RBT, DIM), lambda i: (i, 0)),
      out_shape=jax.ShapeDtypeStruct((N_TAB, DIM), jnp.float32),
  )(*args)


def _score(eh, et, th, tt, rr, ch, ct, alpha):
  def body(eh_r, et_r, th_r, tt_r, rr_r, ch_r, ct_r, a_r, o_ref):
    a0 = a_r[0, 0]
    a1 = a_r[0, 1]
    m = jnp.maximum(a0, a1)
    e0 = jnp.exp(jnp.full((1, DIM), a0 - m, jnp.float32))
    e1 = jnp.exp(jnp.full((1, DIM), a1 - m, jnp.float32))
    w0 = e0 / (e0 + e1)
    w1 = e1 / (e0 + e1)
    cf = ch_r[...].reshape(RB)
    tf = ct_r[...].reshape(RB)
    dh = jnp.where(cf > 0, cf, 1.0)[:, None]
    dt = jnp.where(tf > 0, tf, 1.0)[:, None]
    diff = w0 * (eh_r[...] - et_r[...]) + \
        w1 * (th_r[...] / dh - tt_r[...] / dt) + rr_r[...]
    d2 = jnp.sum(diff * diff, axis=1)
    o_ref[...] = (GAMMA - jnp.sqrt(d2)).reshape(RB // DIM, DIM)

  nb = 16384 // RB
  row = pl.BlockSpec((RB, DIM), lambda i: (i, 0))
  sca = pl.BlockSpec((RB // DIM, DIM), lambda i: (i, 0))
  out = pl.pallas_call(
      body, grid=(nb,),
      in_specs=[row, row, row, row, row, sca, sca,
                pl.BlockSpec((1, 2), lambda i: (0, 0))],
      out_specs=sca,
      out_shape=jax.ShapeDtypeStruct((16384 // DIM, DIM), jnp.float32),
  )(eh, et, th, tt, rr, ch.reshape(16384 // DIM, DIM),
    ct.reshape(16384 // DIM, DIM), alpha.reshape(1, 2))
  return out.reshape(16384)


def _pad_edges(src, dst, share):
  n = src.shape[0]
  pad = NS * share - n
  src = jnp.pad(src, (0, pad))
  dst = jnp.pad(dst, (0, pad), constant_values=BIG)
  return src, dst


def kernel(triples, entity_triangle_index, triangle_tetra_index,
           entity_tetra_index, E, R, W_tri, b_tri, W_tet, b_tet, W_te, b_te,
           fusion_alpha):
  seg1 = _make_seg_sum(4864, 8)
  seg2 = _make_seg_sum(3200, 8)
  gath = _gather_kernel()

  src1, dst1 = _pad_edges(entity_triangle_index[0], entity_triangle_index[1],
                          4864 * 8)
  src2, dst2 = _pad_edges(triangle_tetra_index[0], triangle_tetra_index[1],
                          3200 * 8)
  src3, dst3 = _pad_edges(entity_tetra_index[1], entity_tetra_index[0],
                          3200 * 8)

  E_pad = jnp.pad(E, ((0, N_TAB - N_ENT), (0, 0)))

  s1, c1 = seg1(E_pad, src1, dst1)
  tri = _mean_mm(s1, c1, W_tri.T, b_tri)
  s2, c2 = seg2(tri, src2, dst2)
  proj = _mean_mm(s2, c2, W_tet.T, b_tet, W_te.T, b_te)
  s3, c3 = seg2(proj, src3, dst3)

  h = triples[:, 0]
  r = triples[:, 1]
  t = triples[:, 2]
  eh, et, th, tt, rr, ch, ct = gath(E_pad, R, s3, c3, h, r, t)
  return _score(eh, et, th, tt, rr, ch, ct, fusion_alpha)


# speedup vs baseline: 1.2172x; 1.2172x over previous
import functools

import jax
import jax.numpy as jnp
from jax import lax
from jax.experimental import pallas as pl
from jax.experimental.pallas import tpu as pltpu
from jax.experimental.pallas import tpu_sc as plsc

N_ENT = 100000
DIM = 128
GAMMA = 12.0

NC, NS, L = 2, 16, 16
CHUNK = 8192
N_CHUNKS = 14
N_TAB = CHUNK * N_CHUNKS
CPC = N_CHUNKS // NC
GRP = 128
RPT = CHUNK // NS
RBT = 2048
BIG = 1 << 29
assert CHUNK % 2048 == 0 and N_CHUNKS % 2 == 0 and N_TAB % RBT == 0
assert RPT % GRP == 0 and CHUNK * N_CHUNKS >= N_ENT

_mesh = functools.partial(
    plsc.VectorSubcoreMesh, core_axis_name="c", subcore_axis_name="s",
    num_cores=NC, num_subcores=NS)


def _make_seg_sum(blk, n_blocks):
  share = blk * n_blocks
  g_max = blk // GRP + 2
  assert blk % (2 * L) == 0

  def body(table, src_h, dst_h, sums_o, cnts_o,
           srcblk_a, dstblk_a, srcblk_b, dstblk_b, srcflat, dst2d,
           rows_a, rows_b, ones_v, zcnt, acc, cnt_acc, sem_a, sem_b, sem_e):
    c = lax.axis_index("c")
    s = lax.axis_index("s")
    wid = s * NC + c

    zv = jnp.zeros((L,), jnp.float32)

    def fill_ones(j, _):
      ones_v[pl.ds(j * L, L)] = jnp.full((L,), 1.0, jnp.float32)
      return 0
    lax.fori_loop(0, GRP // L, fill_ones, 0)

    def fill_zcnt(j, _):
      zcnt[pl.ds(j * L, L)] = zv
      return 0
    lax.fori_loop(0, RPT // L, fill_zcnt, 0)

    def zero_acc():
      def fzr(j, _):
        rows_a[j // 8, pl.ds((j % 8) * L, L)] = zv
        return 0
      lax.fori_loop(0, GRP * DIM // L, fzr, 0)
      for j in range(RPT // GRP):
        pltpu.sync_copy(rows_a, acc.at[pl.ds(s * RPT + j * GRP, GRP)])
      pltpu.sync_copy(rows_a.at[pl.ds(0, 8)], acc.at[pl.ds(CHUNK + s * 8, 8)])
      pltpu.sync_copy(zcnt, cnt_acc.at[pl.ds(s * RPT, RPT)])

      @pl.when(s == NS - 1)
      def _():
        pltpu.sync_copy(zcnt.at[pl.ds(0, GRP)],
                        cnt_acc.at[pl.ds(CHUNK, GRP)])

    zero_acc()
    plsc.subcore_barrier()

    iota = lax.iota(jnp.int32, L)
    pad_src = iota + wid * L
    pad_dst = jnp.full((L,), CHUNK, jnp.int32) + s * 8

    def start_g(g, buf, sm):
      pltpu.async_copy(
          table.at[srcflat.at[pl.ds(g * GRP, GRP)]], buf, sm)

    def wait_g(buf, sm):
      pltpu.make_async_copy(
          table.at[srcflat.at[pl.ds(0, GRP)]], buf, sm).wait()

    def scat(g, buf):
      pltpu.sync_copy(buf, acc.at[dst2d.at[g]], add=True)
      pltpu.sync_copy(ones_v, cnt_acc.at[dst2d.at[g]], add=True)

    def flush(ng):
      @pl.when(ng > 0)
      def _():
        start_g(0, rows_a, sem_a)

      def body2(p, _):
        g0 = 2 * p
        g1 = g0 + 1

        @pl.when(g1 < ng)
        def _():
          start_g(g1, rows_b, sem_b)
        wait_g(rows_a, sem_a)
        scat(g0, rows_a)

        @pl.when(g0 + 2 < ng)
        def _():
          start_g(g0 + 2, rows_a, sem_a)

        @pl.when(g1 < ng)
        def _():
          wait_g(rows_b, sem_b)
          scat(g1, rows_b)
        return 0
      lax.fori_loop(0, (ng + 1) // 2, body2, 0)

    for k in range(CPC):
      lo = (k * NC + c) * CHUNK
      hi = lo + CHUNK

      def start_e(b, sb, db):
        base_e = s * share + b * blk
        pltpu.async_copy(src_h.at[pl.ds(base_e, blk)], sb, sem_e)
        pltpu.async_copy(dst_h.at[pl.ds(base_e, blk)], db, sem_e)

      def wait_e(sb, db):
        pltpu.make_async_copy(src_h.at[pl.ds(0, blk)], sb, sem_e).wait()
        pltpu.make_async_copy(dst_h.at[pl.ds(0, blk)], db, sem_e).wait()

      def scan_flush(sb, db, rem):
        def emit(nv, dv, sv):
          m = (dv >= lo) & (dv < hi)
          mi = m.astype(jnp.int32)
          pos = nv + plsc.cumsum(mi) - mi
          plsc.store_scatter(srcflat, [pos], sv, mask=m)
          plsc.store_scatter(dst2d, [pos >> 7, pos & (GRP - 1)], dv - lo,
                             mask=m)
          return nv + plsc.all_reduce_population_count(m)

        def scan2(i2, nv):
          base = i2 * 2 * L
          nv = emit(nv, db[pl.ds(base, L)], sb[pl.ds(base, L)])
          nv = emit(nv, db[pl.ds(base + L, L)], sb[pl.ds(base + L, L)])
          return nv

        nvec = lax.fori_loop(0, blk // (2 * L), scan2,
                             jnp.full((L,), rem, jnp.int32))
        n = jnp.max(nvec)
        ng = n >> 7
        flush(ng)
        for j in range(GRP // L):
          v = srcflat[pl.ds(ng * GRP + j * L, L)]
          srcflat[pl.ds(j * L, L)] = v
          w = dst2d[ng, pl.ds(j * L, L)]
          dst2d[0, pl.ds(j * L, L)] = w
        return n & (GRP - 1)

      start_e(0, srcblk_a, dstblk_a)

      def pair_body(p, rem):
        b1 = 2 * p + 1
        wait_e(srcblk_a, dstblk_a)
        start_e(b1, srcblk_b, dstblk_b)
        rem = scan_flush(srcblk_a, dstblk_a, rem)
        wait_e(srcblk_b, dstblk_b)

        @pl.when(b1 + 1 < n_blocks)
        def _():
          start_e(b1 + 1, srcblk_a, dstblk_a)
        rem = scan_flush(srcblk_b, dstblk_b, rem)
        return rem

      rem = lax.fori_loop(0, n_blocks // 2, pair_body, jnp.int32(0))

      for j in range(GRP // L):
        idxp = rem + j * L + iota
        plsc.store_scatter(srcflat, [idxp], pad_src)
        plsc.store_scatter(dst2d, [idxp >> 7, idxp & (GRP - 1)], pad_dst)

      @pl.when(rem > 0)
      def _():
        start_g(0, rows_a, sem_a)
        wait_g(rows_a, sem_a)
        scat(0, rows_a)

      plsc.subcore_barrier()
      pltpu.sync_copy(acc.at[pl.ds(s * RPT, RPT)],
                      sums_o.at[pl.ds(lo + s * RPT, RPT)])
      pltpu.sync_copy(cnt_acc.at[pl.ds(s * RPT, RPT)],
                      cnts_o.at[pl.ds(lo + s * RPT, RPT)])
      zero_acc()
      plsc.subcore_barrier()

  return pl.kernel(
      body,
      out_type=(jax.ShapeDtypeStruct((N_TAB, DIM), jnp.float32),
                jax.ShapeDtypeStruct((N_TAB,), jnp.float32)),
      mesh=_mesh(),
      compiler_params=pltpu.CompilerParams(needs_layout_passes=False),
      scratch_types=[
          pltpu.VMEM((blk,), jnp.int32),
          pltpu.VMEM((blk,), jnp.int32),
          pltpu.VMEM((blk,), jnp.int32),
          pltpu.VMEM((blk,), jnp.int32),
          pltpu.VMEM((blk + 2 * GRP,), jnp.int32),
          pltpu.VMEM((g_max, GRP), jnp.int32),
          pltpu.VMEM((GRP, DIM), jnp.float32),
          pltpu.VMEM((GRP, DIM), jnp.float32),
          pltpu.VMEM((GRP,), jnp.float32),
          pltpu.VMEM((RPT,), jnp.float32),
          pltpu.VMEM_SHARED((CHUNK + GRP, DIM), jnp.float32),
          pltpu.VMEM_SHARED((CHUNK + GRP,), jnp.float32),
          pltpu.SemaphoreType.DMA,
          pltpu.SemaphoreType.DMA,
          pltpu.SemaphoreType.DMA,
      ],
  )


def _gather_kernel():
  nt = 16384 // (NC * NS)

  def body(e_h, r_h, s3_h, c3_h, hh, rh, th,
           eh_o, et_o, th_o, tt_o, rr_o, ch_o, ct_o,
           hbuf, rbuf, tbuf, rows_v, crow, sem):
    c = lax.axis_index("c")
    s = lax.axis_index("s")
    wid = s * NC + c
    base = wid * nt
    pltpu.sync_copy(hh.at[pl.ds(base, nt)], hbuf)
    pltpu.sync_copy(rh.at[pl.ds(base, nt)], rbuf)
    pltpu.sync_copy(th.at[pl.ds(base, nt)], tbuf)
    for g in range(nt // GRP):
      sl = pl.ds(base + g * GRP, GRP)
      gi = pl.ds(g * GRP, GRP)
      for tab, idxb, out in ((e_h, hbuf, eh_o), (e_h, tbuf, et_o),
                             (s3_h, hbuf, th_o), (s3_h, tbuf, tt_o),
                             (r_h, rbuf, rr_o)):
        pltpu.async_copy(tab.at[idxb.at[gi]], rows_v, sem).wait()
        pltpu.sync_copy(rows_v, out.at[sl])
      for idxb, out in ((hbuf, ch_o), (tbuf, ct_o)):
        pltpu.async_copy(c3_h.at[idxb.at[gi]], crow, sem).wait()
        pltpu.sync_copy(crow, out.at[sl])

  f32 = jnp.float32
  return pl.kernel(
      body,
      out_type=(jax.ShapeDtypeStruct((16384, DIM), f32),) * 5
      + (jax.ShapeDtypeStruct((16384,), f32),) * 2,
      mesh=_mesh(),
      compiler_params=pltpu.CompilerParams(needs_layout_passes=False),
      scratch_types=[
          pltpu.VMEM((nt,), jnp.int32),
          pltpu.VMEM((nt,), jnp.int32),
          pltpu.VMEM((nt,), jnp.int32),
          pltpu.VMEM((GRP, DIM), f32),
          pltpu.VMEM((GRP,), f32),
          pltpu.SemaphoreType.DMA,
      ],
  )


RB = 2048


def _mean_mm(sums, cnts, wt1, b1, wt2=None, b2=None):
  two = wt2 is not None

  def body(s_ref, c_ref, w1_ref, b1_ref, *rest):
    if two:
      w2_ref, b2_ref, o_ref = rest
    else:
      (o_ref,) = rest
    cnt = c_ref[...].reshape(RBT)
    den = jnp.where(cnt > 0, cnt, 1.0)
    x = s_ref[...] / den[:, None]
    y = lax.dot_general(x, w1_ref[...], (((1,), (0,)), ((), ())),
                        preferred_element_type=jnp.float32,
                        precision=lax.Precision.HIGHEST)
    y = jnp.maximum(y + b1_ref[...], 0.0)
    if two:
      y = lax.dot_general(y, w2_ref[...], (((1,), (0,)), ((), ())),
                          preferred_element_type=jnp.float32,
                          precision=lax.Precision.HIGHEST) + b2_ref[...]
    o_ref[...] = y

  n_blk = N_TAB // RBT
  in_specs = [
      pl.BlockSpec((RBT, DIM), lambda i: (i, 0)),
      pl.BlockSpec((RBT // DIM, DIM), lambda i: (i, 0)),
      pl.BlockSpec((DIM, DIM), lambda i: (0, 0)),
      pl.BlockSpec((1, DIM), lambda i: (0, 0)),
  ]
  args = [sums, cnts.reshape(N_TAB // DIM, DIM), wt1, b1.reshape(1, DIM)]
  if two:
    in_specs += [pl.BlockSpec((DIM, DIM), lambda i: (0, 0)),
                 pl.BlockSpec((1, DIM), lambda i: (0, 0))]
    args += [wt2, b2.reshape(1, DIM)]
  return pl.pallas_call(
      body, grid=(n_blk,), in_specs=in_specs,
      out_specs=pl.BlockSpec((RBT, DIM), lambda i: (i, 0)),
      out_shape=jax.ShapeDtypeStruct((N_TAB, DIM), jnp.float32),
  )(*args)


def _score(eh, et, th, tt, rr, ch, ct, alpha):
  def body(eh_r, et_r, th_r, tt_r, rr_r, ch_r, ct_r, a_r, o_ref):
    a0 = a_r[0, 0]
    a1 = a_r[0, 1]
    m = jnp.maximum(a0, a1)
    e0 = jnp.exp(jnp.full((1, DIM), a0 - m, jnp.float32))
    e1 = jnp.exp(jnp.full((1, DIM), a1 - m, jnp.float32))
    w0 = e0 / (e0 + e1)
    w1 = e1 / (e0 + e1)
    cf = ch_r[...].reshape(RB)
    tf = ct_r[...].reshape(RB)
    dh = jnp.where(cf > 0, cf, 1.0)[:, None]
    dt = jnp.where(tf > 0, tf, 1.0)[:, None]
    diff = w0 * (eh_r[...] - et_r[...]) + \
        w1 * (th_r[...] / dh - tt_r[...] / dt) + rr_r[...]
    d2 = jnp.sum(diff * diff, axis=1)
    o_ref[...] = (GAMMA - jnp.sqrt(d2)).reshape(RB // DIM, DIM)

  nb = 16384 // RB
  row = pl.BlockSpec((RB, DIM), lambda i: (i, 0))
  sca = pl.BlockSpec((RB // DIM, DIM), lambda i: (i, 0))
  out = pl.pallas_call(
      body, grid=(nb,),
      in_specs=[row, row, row, row, row, sca, sca,
                pl.BlockSpec((1, 2), lambda i: (0, 0))],
      out_specs=sca,
      out_shape=jax.ShapeDtypeStruct((16384 // DIM, DIM), jnp.float32),
  )(eh, et, th, tt, rr, ch.reshape(16384 // DIM, DIM),
    ct.reshape(16384 // DIM, DIM), alpha.reshape(1, 2))
  return out.reshape(16384)


def _pad_edges(src, dst, share):
  n = src.shape[0]
  pad = NS * share - n
  src = jnp.pad(src, (0, pad))
  dst = jnp.pad(dst, (0, pad), constant_values=BIG)
  return src, dst


def kernel(triples, entity_triangle_index, triangle_tetra_index,
           entity_tetra_index, E, R, W_tri, b_tri, W_tet, b_tet, W_te, b_te,
           fusion_alpha):
  seg1 = _make_seg_sum(4704, 8)
  seg2 = _make_seg_sum(3200, 8)
  gath = _gather_kernel()

  src1, dst1 = _pad_edges(entity_triangle_index[0], entity_triangle_index[1],
                          4704 * 8)
  src2, dst2 = _pad_edges(triangle_tetra_index[0], triangle_tetra_index[1],
                          3200 * 8)
  src3, dst3 = _pad_edges(entity_tetra_index[1], entity_tetra_index[0],
                          3200 * 8)

  E_pad = jnp.pad(E, ((0, N_TAB - N_ENT), (0, 0)))

  s1, c1 = seg1(E_pad, src1, dst1)
  tri = _mean_mm(s1, c1, W_tri.T, b_tri)
  s2, c2 = seg2(tri, src2, dst2)
  proj = _mean_mm(s2, c2, W_tet.T, b_tet, W_te.T, b_te)
  s3, c3 = seg2(proj, src3, dst3)

  h = triples[:, 0]
  r = triples[:, 1]
  t = triples[:, 2]
  eh, et, th, tt, rr, ch, ct = gath(E_pad, R, s3, c3, h, r, t)
  return _score(eh, et, th, tt, rr, ch, ct, fusion_alpha)


# speedup vs baseline: 1.2200x; 1.0023x over previous
import functools

import jax
import jax.numpy as jnp
from jax import lax
from jax.experimental import pallas as pl
from jax.experimental.pallas import tpu as pltpu
from jax.experimental.pallas import tpu_sc as plsc

N_ENT = 100000
DIM = 128
GAMMA = 12.0

NC, NS, L = 2, 16, 16
CHUNK = 8192
N_CHUNKS = 14
N_TAB = CHUNK * N_CHUNKS
CPC = N_CHUNKS // NC
GRP = 128
RPT = CHUNK // NS
RBT = 2048
BIG = 1 << 29
assert CHUNK % 2048 == 0 and N_CHUNKS % 2 == 0 and N_TAB % RBT == 0
assert RPT % GRP == 0 and CHUNK * N_CHUNKS >= N_ENT

_mesh = functools.partial(
    plsc.VectorSubcoreMesh, core_axis_name="c", subcore_axis_name="s",
    num_cores=NC, num_subcores=NS)


def _make_seg_sum(blk, n_blocks):
  share = blk * n_blocks
  g_max = blk // GRP + 2
  assert blk % (4 * L) == 0

  def body(table, src_h, dst_h, sums_o, cnts_o,
           srcblk_a, dstblk_a, srcblk_b, dstblk_b, srcflat, dst2d,
           rows_a, rows_b, ones_v, zcnt, acc, cnt_acc, sem_a, sem_b, sem_e):
    c = lax.axis_index("c")
    s = lax.axis_index("s")
    wid = s * NC + c

    zv = jnp.zeros((L,), jnp.float32)

    def fill_ones(j, _):
      ones_v[pl.ds(j * L, L)] = jnp.full((L,), 1.0, jnp.float32)
      return 0
    lax.fori_loop(0, GRP // L, fill_ones, 0)

    def fill_zcnt(j, _):
      zcnt[pl.ds(j * L, L)] = zv
      return 0
    lax.fori_loop(0, RPT // L, fill_zcnt, 0)

    def zero_acc():
      def fzr(j, _):
        rows_a[j // 8, pl.ds((j % 8) * L, L)] = zv
        return 0
      lax.fori_loop(0, GRP * DIM // L, fzr, 0)
      for j in range(RPT // GRP):
        pltpu.sync_copy(rows_a, acc.at[pl.ds(s * RPT + j * GRP, GRP)])
      pltpu.sync_copy(rows_a.at[pl.ds(0, 8)], acc.at[pl.ds(CHUNK + s * 8, 8)])
      pltpu.sync_copy(zcnt, cnt_acc.at[pl.ds(s * RPT, RPT)])

      @pl.when(s == NS - 1)
      def _():
        pltpu.sync_copy(zcnt.at[pl.ds(0, GRP)],
                        cnt_acc.at[pl.ds(CHUNK, GRP)])

    zero_acc()
    plsc.subcore_barrier()

    iota = lax.iota(jnp.int32, L)
    pad_src = iota + wid * L
    pad_dst = jnp.full((L,), CHUNK, jnp.int32) + s * 8

    def start_g(g, buf, sm):
      pltpu.async_copy(
          table.at[srcflat.at[pl.ds(g * GRP, GRP)]], buf, sm)

    def wait_g(buf, sm):
      pltpu.make_async_copy(
          table.at[srcflat.at[pl.ds(0, GRP)]], buf, sm).wait()

    def scat(g, buf):
      pltpu.sync_copy(buf, acc.at[dst2d.at[g]], add=True)
      pltpu.sync_copy(ones_v, cnt_acc.at[dst2d.at[g]], add=True)

    def flush(ng):
      @pl.when(ng > 0)
      def _():
        start_g(0, rows_a, sem_a)

      def body2(p, _):
        g0 = 2 * p
        g1 = g0 + 1

        @pl.when(g1 < ng)
        def _():
          start_g(g1, rows_b, sem_b)
        wait_g(rows_a, sem_a)
        scat(g0, rows_a)

        @pl.when(g0 + 2 < ng)
        def _():
          start_g(g0 + 2, rows_a, sem_a)

        @pl.when(g1 < ng)
        def _():
          wait_g(rows_b, sem_b)
          scat(g1, rows_b)
        return 0
      lax.fori_loop(0, (ng + 1) // 2, body2, 0)

    for k in range(CPC):
      lo = (k * NC + c) * CHUNK
      hi = lo + CHUNK

      def start_e(b, sb, db):
        base_e = s * share + b * blk
        pltpu.async_copy(src_h.at[pl.ds(base_e, blk)], sb, sem_e)
        pltpu.async_copy(dst_h.at[pl.ds(base_e, blk)], db, sem_e)

      def wait_e(sb, db):
        pltpu.make_async_copy(src_h.at[pl.ds(0, blk)], sb, sem_e).wait()
        pltpu.make_async_copy(dst_h.at[pl.ds(0, blk)], db, sem_e).wait()

      def scan_flush(sb, db, rem):
        def emit(nv, dv, sv):
          m = (dv >= lo) & (dv < hi)
          mi = m.astype(jnp.int32)
          pos = nv + plsc.cumsum(mi) - mi
          plsc.store_scatter(srcflat, [pos], sv, mask=m)
          plsc.store_scatter(dst2d, [pos >> 7, pos & (GRP - 1)], dv - lo,
                             mask=m)
          return nv + plsc.all_reduce_population_count(m)

        def scan4(i4, nv):
          base = i4 * 4 * L
          for u in range(4):
            nv = emit(nv, db[pl.ds(base + u * L, L)],
                      sb[pl.ds(base + u * L, L)])
          return nv

        nvec = lax.fori_loop(0, blk // (4 * L), scan4,
                             jnp.full((L,), rem, jnp.int32))
        n = jnp.max(nvec)
        ng = n >> 7
        flush(ng)
        for j in range(GRP // L):
          v = srcflat[pl.ds(ng * GRP + j * L, L)]
          srcflat[pl.ds(j * L, L)] = v
          w = dst2d[ng, pl.ds(j * L, L)]
          dst2d[0, pl.ds(j * L, L)] = w
        return n & (GRP - 1)

      start_e(0, srcblk_a, dstblk_a)

      def pair_body(p, rem):
        b1 = 2 * p + 1
        wait_e(srcblk_a, dstblk_a)
        start_e(b1, srcblk_b, dstblk_b)
        rem = scan_flush(srcblk_a, dstblk_a, rem)
        wait_e(srcblk_b, dstblk_b)

        @pl.when(b1 + 1 < n_blocks)
        def _():
          start_e(b1 + 1, srcblk_a, dstblk_a)
        rem = scan_flush(srcblk_b, dstblk_b, rem)
        return rem

      rem = lax.fori_loop(0, n_blocks // 2, pair_body, jnp.int32(0))

      for j in range(GRP // L):
        idxp = rem + j * L + iota
        plsc.store_scatter(srcflat, [idxp], pad_src)
        plsc.store_scatter(dst2d, [idxp >> 7, idxp & (GRP - 1)], pad_dst)

      @pl.when(rem > 0)
      def _():
        start_g(0, rows_a, sem_a)
        wait_g(rows_a, sem_a)
        scat(0, rows_a)

      plsc.subcore_barrier()
      pltpu.sync_copy(acc.at[pl.ds(s * RPT, RPT)],
                      sums_o.at[pl.ds(lo + s * RPT, RPT)])
      pltpu.sync_copy(cnt_acc.at[pl.ds(s * RPT, RPT)],
                      cnts_o.at[pl.ds(lo + s * RPT, RPT)])
      zero_acc()
      plsc.subcore_barrier()

  return pl.kernel(
      body,
      out_type=(jax.ShapeDtypeStruct((N_TAB, DIM), jnp.float32),
                jax.ShapeDtypeStruct((N_TAB,), jnp.float32)),
      mesh=_mesh(),
      compiler_params=pltpu.CompilerParams(needs_layout_passes=False),
      scratch_types=[
          pltpu.VMEM((blk,), jnp.int32),
          pltpu.VMEM((blk,), jnp.int32),
          pltpu.VMEM((blk,), jnp.int32),
          pltpu.VMEM((blk,), jnp.int32),
          pltpu.VMEM((blk + 2 * GRP,), jnp.int32),
          pltpu.VMEM((g_max, GRP), jnp.int32),
          pltpu.VMEM((GRP, DIM), jnp.float32),
          pltpu.VMEM((GRP, DIM), jnp.float32),
          pltpu.VMEM((GRP,), jnp.float32),
          pltpu.VMEM((RPT,), jnp.float32),
          pltpu.VMEM_SHARED((CHUNK + GRP, DIM), jnp.float32),
          pltpu.VMEM_SHARED((CHUNK + GRP,), jnp.float32),
          pltpu.SemaphoreType.DMA,
          pltpu.SemaphoreType.DMA,
          pltpu.SemaphoreType.DMA,
      ],
  )


def _gather_kernel():
  nt = 16384 // (NC * NS)

  def body(e_h, r_h, s3_h, c3_h, hh, rh, th,
           eh_o, et_o, th_o, tt_o, rr_o, ch_o, ct_o,
           hbuf, rbuf, tbuf, rows_v, crow, sem):
    c = lax.axis_index("c")
    s = lax.axis_index("s")
    wid = s * NC + c
    base = wid * nt
    pltpu.sync_copy(hh.at[pl.ds(base, nt)], hbuf)
    pltpu.sync_copy(rh.at[pl.ds(base, nt)], rbuf)
    pltpu.sync_copy(th.at[pl.ds(base, nt)], tbuf)
    for g in range(nt // GRP):
      sl = pl.ds(base + g * GRP, GRP)
      gi = pl.ds(g * GRP, GRP)
      for tab, idxb, out in ((e_h, hbuf, eh_o), (e_h, tbuf, et_o),
                             (s3_h, hbuf, th_o), (s3_h, tbuf, tt_o),
                             (r_h, rbuf, rr_o)):
        pltpu.async_copy(tab.at[idxb.at[gi]], rows_v, sem).wait()
        pltpu.sync_copy(rows_v, out.at[sl])
      for idxb, out in ((hbuf, ch_o), (tbuf, ct_o)):
        pltpu.async_copy(c3_h.at[idxb.at[gi]], crow, sem).wait()
        pltpu.sync_copy(crow, out.at[sl])

  f32 = jnp.float32
  return pl.kernel(
      body,
      out_type=(jax.ShapeDtypeStruct((16384, DIM), f32),) * 5
      + (jax.ShapeDtypeStruct((16384,), f32),) * 2,
      mesh=_mesh(),
      compiler_params=pltpu.CompilerParams(needs_layout_passes=False),
      scratch_types=[
          pltpu.VMEM((nt,), jnp.int32),
          pltpu.VMEM((nt,), jnp.int32),
          pltpu.VMEM((nt,), jnp.int32),
          pltpu.VMEM((GRP, DIM), f32),
          pltpu.VMEM((GRP,), f32),
          pltpu.SemaphoreType.DMA,
      ],
  )


RB = 2048


def _mean_mm(sums, cnts, wt1, b1, wt2=None, b2=None):
  two = wt2 is not None

  def body(s_ref, c_ref, w1_ref, b1_ref, *rest):
    if two:
      w2_ref, b2_ref, o_ref = rest
    else:
      (o_ref,) = rest
    cnt = c_ref[...].reshape(RBT)
    den = jnp.where(cnt > 0, cnt, 1.0)
    x = s_ref[...] / den[:, None]
    y = lax.dot_general(x, w1_ref[...], (((1,), (0,)), ((), ())),
                        preferred_element_type=jnp.float32,
                        precision=lax.Precision.HIGHEST)
    y = jnp.maximum(y + b1_ref[...], 0.0)
    if two:
      y = lax.dot_general(y, w2_ref[...], (((1,), (0,)), ((), ())),
                          preferred_element_type=jnp.float32,
                          precision=lax.Precision.HIGHEST) + b2_ref[...]
    o_ref[...] = y

  n_blk = N_TAB // RBT
  in_specs = [
      pl.BlockSpec((RBT, DIM), lambda i: (i, 0)),
      pl.BlockSpec((RBT // DIM, DIM), lambda i: (i, 0)),
      pl.BlockSpec((DIM, DIM), lambda i: (0, 0)),
      pl.BlockSpec((1, DIM), lambda i: (0, 0)),
  ]
  args = [sums, cnts.reshape(N_TAB // DIM, DIM), wt1, b1.reshape(1, DIM)]
  if two:
    in_specs += [pl.BlockSpec((DIM, DIM), lambda i: (0, 0)),
                 pl.BlockSpec((1, DIM), lambda i: (0, 0))]
    args += [wt2, b2.reshape(1, DIM)]
  return pl.pallas_call(
      body, grid=(n_blk,), in_specs=in_specs,
      out_specs=pl.BlockSpec((RBT, DIM), lambda i: (i, 0)),
      out_shape=jax.ShapeDtypeStruct((N_TAB, DIM), jnp.float32),
  )(*args)


def _score(eh, et, th, tt, rr, ch, ct, alpha):
  def body(eh_r, et_r, th_r, tt_r, rr_r, ch_r, ct_r, a_r, o_ref):
    a0 = a_r[0, 0]
    a1 = a_r[0, 1]
    m = jnp.maximum(a0, a1)
    e0 = jnp.exp(jnp.full((1, DIM), a0 - m, jnp.float32))
    e1 = jnp.exp(jnp.full((1, DIM), a1 - m, jnp.float32))
    w0 = e0 / (e0 + e1)
    w1 = e1 / (e0 + e1)
    cf = ch_r[...].reshape(RB)
    tf = ct_r[...].reshape(RB)
    dh = jnp.where(cf > 0, cf, 1.0)[:, None]
    dt = jnp.where(tf > 0, tf, 1.0)[:, None]
    diff = w0 * (eh_r[...] - et_r[...]) + \
        w1 * (th_r[...] / dh - tt_r[...] / dt) + rr_r[...]
    d2 = jnp.sum(diff * diff, axis=1)
    o_ref[...] = (GAMMA - jnp.sqrt(d2)).reshape(RB // DIM, DIM)

  nb = 16384 // RB
  row = pl.BlockSpec((RB, DIM), lambda i: (i, 0))
  sca = pl.BlockSpec((RB // DIM, DIM), lambda i: (i, 0))
  out = pl.pallas_call(
      body, grid=(nb,),
      in_specs=[row, row, row, row, row, sca, sca,
                pl.BlockSpec((1, 2), lambda i: (0, 0))],
      out_specs=sca,
      out_shape=jax.ShapeDtypeStruct((16384 // DIM, DIM), jnp.float32),
  )(eh, et, th, tt, rr, ch.reshape(16384 // DIM, DIM),
    ct.reshape(16384 // DIM, DIM), alpha.reshape(1, 2))
  return out.reshape(16384)


def _pad_edges(src, dst, share):
  n = src.shape[0]
  pad = NS * share - n
  src = jnp.pad(src, (0, pad))
  dst = jnp.pad(dst, (0, pad), constant_values=BIG)
  return src, dst


def kernel(triples, entity_triangle_index, triangle_tetra_index,
           entity_tetra_index, E, R, W_tri, b_tri, W_tet, b_tet, W_te, b_te,
           fusion_alpha):
  seg1 = _make_seg_sum(4736, 8)
  seg2 = _make_seg_sum(3200, 8)
  gath = _gather_kernel()

  src1, dst1 = _pad_edges(entity_triangle_index[0], entity_triangle_index[1],
                          4736 * 8)
  src2, dst2 = _pad_edges(triangle_tetra_index[0], triangle_tetra_index[1],
                          3200 * 8)
  src3, dst3 = _pad_edges(entity_tetra_index[1], entity_tetra_index[0],
                          3200 * 8)

  E_pad = jnp.pad(E, ((0, N_TAB - N_ENT), (0, 0)))

  s1, c1 = seg1(E_pad, src1, dst1)
  tri = _mean_mm(s1, c1, W_tri.T, b_tri)
  s2, c2 = seg2(tri, src2, dst2)
  proj = _mean_mm(s2, c2, W_tet.T, b_tet, W_te.T, b_te)
  s3, c3 = seg2(proj, src3, dst3)

  h = triples[:, 0]
  r = triples[:, 1]
  t = triples[:, 2]
  eh, et, th, tt, rr, ch, ct = gath(E_pad, R, s3, c3, h, r, t)
  return _score(eh, et, th, tt, rr, ch, ct, fusion_alpha)
